# Initial kernel scaffold; baseline (speedup 1.0000x reference)
import jax
import jax.numpy as jnp
from jax.experimental import pallas as pl


def kernel(x, a, b, c3, c4, c5, S, idx3, idx4, idx5):
    raise NotImplementedError("write your pallas kernel here")


# speedup vs baseline: 6.5866x; 6.5866x over previous
import numpy as np
import jax
import jax.numpy as jnp
from jax.experimental import pallas as pl
from jax.experimental.pallas import tpu as pltpu

_R = 512
_TRIU_I, _TRIU_J = np.triu_indices(8, k=1)


def _poly_body(s_ref, *refs):
    xblks = refs[:8]
    e_ref, g_ref, out_ref = refs[8], refs[9], refs[10]
    xsa = jnp.zeros((_R, 16), dtype=jnp.float32)
    for j in range(8):
        xsa = xsa + jnp.dot(xblks[j][...], e_ref[j],
                            preferred_element_type=jnp.float32)
    lane16 = jax.lax.broadcasted_iota(jnp.int32, (_R, 16), 1)
    xsa = xsa + jnp.where(lane16 == 8, 1.0, 0.0)
    v = jnp.dot(xsa, g_ref[...], preferred_element_type=jnp.float32)
    p = (v[:, 0:128] * v[:, 128:256] * v[:, 256:384]
         * v[:, 384:512] * v[:, 512:640])
    out_ref[...] = jnp.sum(p, axis=1, keepdims=True)


def kernel(x, a, b, c3, c4, c5, S, idx3, idx4, idx5):
    B, D = x.shape
    s32 = S.astype(jnp.int32)
    i3 = idx3.astype(jnp.int32)
    i4 = idx4.astype(jnp.int32)
    i5 = idx5.astype(jnp.int32)

    lj = s32 % 128
    E = ((jnp.arange(128, dtype=jnp.int32)[None, :, None] == lj[:, None, None])
         & (jnp.arange(16, dtype=jnp.int32)[None, None, :]
            == jnp.arange(8, dtype=jnp.int32)[:, None, None])
         ).astype(jnp.float32)

    ones8 = jnp.full((8,), 8, jnp.int32)
    ones28 = jnp.full((28,), 8, jnp.int32)
    ones12 = jnp.full((12,), 8, jnp.int32)
    ones4 = jnp.full((4,), 8, jnp.int32)
    ti = jnp.asarray(_TRIU_I, jnp.int32)
    tj = jnp.asarray(_TRIU_J, jnp.int32)
    lin = jnp.arange(8, dtype=jnp.int32)
    feat = [
        jnp.concatenate([lin, ti, i3[:, 0], i4[:, 0], i5[:, 0]]),
        jnp.concatenate([ones8, tj, i3[:, 1], i4[:, 1], i5[:, 1]]),
        jnp.concatenate([ones8, ones28, i3[:, 2], i4[:, 2], i5[:, 2]]),
        jnp.concatenate([ones8, ones28, ones12, i4[:, 3], i5[:, 3]]),
        jnp.concatenate([ones8, ones28, ones12, jnp.full((8,), 8, jnp.int32),
                         i5[:, 4]]),
    ]
    coef = jnp.concatenate([a, b[ti, tj], c3, c4, c5])

    gtiles = []
    ar16 = jnp.arange(16, dtype=jnp.int32)
    for k in range(5):
        oh = (feat[k][:, None] == ar16[None, :]).astype(jnp.float32)
        if k == 0:
            oh = oh * coef[:, None]
        gk = jnp.pad(oh.T, ((0, 0), (0, 68)))
        gtiles.append(gk)
    G = jnp.concatenate(gtiles, axis=1)

    nb = B // _R
    grid_spec = pltpu.PrefetchScalarGridSpec(
        num_scalar_prefetch=1,
        grid=(nb,),
        in_specs=(
            [pl.BlockSpec((_R, 128), (lambda i, s, j=j: (i, s[j] // 128)))
             for j in range(8)]
            + [pl.BlockSpec((8, 128, 16), lambda i, s: (0, 0, 0)),
               pl.BlockSpec((16, 640), lambda i, s: (0, 0))]
        ),
        out_specs=pl.BlockSpec((_R, 1), lambda i, s: (i, 0)),
    )
    out = pl.pallas_call(
        _poly_body,
        grid_spec=grid_spec,
        out_shape=jax.ShapeDtypeStruct((B, 1), jnp.float32),
        compiler_params=pltpu.CompilerParams(
            dimension_semantics=("arbitrary",),
        ),
    )(s32, x, x, x, x, x, x, x, x, E, G)
    return out.reshape(B)
